# baseline (speedup 1.0000x reference)
import functools

import jax
import jax.numpy as jnp
from jax import lax
from jax.experimental import pallas as pl
from jax.experimental.pallas import tpu as pltpu
from jax.experimental.pallas import tpu_sc as plsc

NUM_MESH = 10242
FEAT = 128
HFEAT = FEAT // 2
CNTW = 16
N_PAD = 10368
NC = 2
NS = 16
EDGES = 320000
TROWS = 2 * NUM_MESH
CHUNK = 80
NBUF = 2
E_PAD = 320000
E_PER_T = E_PAD // NS
N_CHUNKS = E_PER_T // CHUNK
N_OUTER = N_CHUNKS // NBUF
ROWS_PER_TILE = N_PAD // NS
STG = 24


def _sc_body(table_hbm, src_hbm, dst_hbm,
             psum_hbm, cnt_hbm,
             src_v, dst_v, idx_v, rows_v, ones_v, stg_v, cstg_v,
             acc_sh, cnt_sh, sem_in, sem_g):
    c = lax.axis_index("c")
    s = lax.axis_index("s")
    row0 = s * ROWS_PER_TILE
    core0 = c == 0
    core1 = c == 1

    zeros16 = jnp.zeros((16,), jnp.float32)
    ones16 = jnp.ones((16,), jnp.float32)

    def fill_stg(i, carry):
        for j in range(HFEAT // 16):
            stg_v[i, pl.ds(j * 16, 16)] = zeros16
        return carry
    lax.fori_loop(0, STG, fill_stg, 0)

    def fill_cstg(i, carry):
        cstg_v[i, pl.ds(0, 16)] = zeros16
        return carry
    lax.fori_loop(0, STG, fill_cstg, 0)

    def fill_ones(i, carry):
        ones_v[i, pl.ds(0, 16)] = ones16
        return carry
    lax.fori_loop(0, CHUNK, fill_ones, 0)

    for q in range(ROWS_PER_TILE // STG):
        pltpu.sync_copy(stg_v, acc_sh.at[pl.ds(row0 + q * STG, STG)])
        pltpu.sync_copy(cstg_v, cnt_sh.at[pl.ds(row0 + q * STG, STG)])

    plsc.subcore_barrier()

    base_t = s * E_PER_T

    def issue_loads(k, b):
        base = pl.multiple_of(base_t + k * CHUNK, 8)
        pltpu.async_copy(src_hbm.at[pl.ds(base, CHUNK)], src_v.at[b], sem_in[b])
        pltpu.async_copy(dst_hbm.at[pl.ds(base, CHUNK)], dst_v.at[b], sem_in[b])

    def wait_loads(b):
        pltpu.make_async_copy(src_hbm.at[pl.ds(0, CHUNK)], src_v.at[b],
                              sem_in[b]).wait()
        pltpu.make_async_copy(dst_hbm.at[pl.ds(0, CHUNK)], dst_v.at[b],
                              sem_in[b]).wait()

    half_base = c * (TROWS + 1)

    def transform_and_gather(b):
        for j in range(CHUNK // 16):
            sl = pl.ds(j * 16, 16)
            idx_v[b, sl] = src_v[b, sl] * 2 + half_base
        pltpu.async_copy(table_hbm.at[idx_v.at[b]], rows_v.at[b], sem_g[b])

    def wait_gather(b):
        pltpu.make_async_copy(table_hbm.at[idx_v.at[b]], rows_v.at[b],
                              sem_g[b]).wait()

    issue_loads(0, 0)
    issue_loads(1, 1)
    wait_loads(0)
    transform_and_gather(0)

    def outer_body(k0, carry):
        for b in range(NBUF):
            k = k0 * NBUF + b
            nb = (b + 1) % NBUF
            if b < NBUF - 1:
                wait_loads(nb)
                transform_and_gather(nb)
            else:
                @pl.when(k0 < N_OUTER - 1)
                def _():
                    wait_loads(nb)
                    transform_and_gather(nb)

            wait_gather(b)
            pltpu.sync_copy(rows_v.at[b], acc_sh.at[dst_v.at[b]], add=True)

            @pl.when(core0 if b == 0 else core1)
            def _():
                pltpu.sync_copy(ones_v, cnt_sh.at[dst_v.at[b]], add=True)

            @pl.when(k0 < N_OUTER - 1)
            def _():
                issue_loads(k + NBUF, b)
        return carry

    lax.fori_loop(0, N_OUTER, outer_body, 0)

    plsc.subcore_barrier()

    out0 = c * N_PAD + row0
    for q in range(ROWS_PER_TILE // STG):
        pltpu.sync_copy(acc_sh.at[pl.ds(row0 + q * STG, STG)], stg_v)
        pltpu.sync_copy(stg_v, psum_hbm.at[pl.ds(out0 + q * STG, STG)])
        pltpu.sync_copy(cnt_sh.at[pl.ds(row0 + q * STG, STG)], cstg_v)
        pltpu.sync_copy(cstg_v, cnt_hbm.at[pl.ds(out0 + q * STG, STG)])


def _make_sc_call():
    mesh = plsc.VectorSubcoreMesh(core_axis_name="c", subcore_axis_name="s")
    return functools.partial(
        pl.kernel,
        mesh=mesh,
        compiler_params=pltpu.CompilerParams(use_tc_tiling_on_sc=False),
        out_type=(
            jax.ShapeDtypeStruct((NC * N_PAD, HFEAT), jnp.float32),
            jax.ShapeDtypeStruct((NC * N_PAD, CNTW), jnp.float32),
        ),
        scratch_types=[
            pltpu.VMEM((NBUF, CHUNK), jnp.int32),
            pltpu.VMEM((NBUF, CHUNK), jnp.int32),
            pltpu.VMEM((NBUF, CHUNK), jnp.int32),
            pltpu.VMEM((NBUF, CHUNK, HFEAT), jnp.float32),
            pltpu.VMEM((CHUNK, CNTW), jnp.float32),
            pltpu.VMEM((STG, HFEAT), jnp.float32),
            pltpu.VMEM((STG, CNTW), jnp.float32),
            pltpu.VMEM_SHARED((N_PAD, HFEAT), jnp.float32),
            pltpu.VMEM_SHARED((N_PAD, CNTW), jnp.float32),
            [pltpu.SemaphoreType.DMA] * NBUF,
            [pltpu.SemaphoreType.DMA] * NBUF,
        ],
    )(_sc_body)


def _combine_body(psum_ref, cnt_ref, out_ref):
    total = jnp.concatenate((psum_ref[0], psum_ref[1]), axis=1)
    counts = cnt_ref[0, :, 0] + cnt_ref[1, :, 0]
    out_ref[...] = total / jnp.maximum(counts, 1.0)[:, None]


def _combine(psum, cnt):
    grid = N_PAD // 128
    return pl.pallas_call(
        _combine_body,
        grid=(grid,),
        in_specs=[
            pl.BlockSpec((NC, 128, HFEAT), lambda i: (0, i, 0)),
            pl.BlockSpec((NC, 128, CNTW), lambda i: (0, i, 0)),
        ],
        out_specs=pl.BlockSpec((128, FEAT), lambda i: (i, 0)),
        out_shape=jax.ShapeDtypeStruct((N_PAD, FEAT), jnp.float32),
    )(psum, cnt)


def kernel(grid_node_features, edge_index):
    feats = grid_node_features[0]
    used = feats[:NUM_MESH].reshape(-1, HFEAT)
    table2 = jnp.broadcast_to(used[None], (NC, TROWS, HFEAT)).reshape(-1, HFEAT)
    eidx = edge_index[0].astype(jnp.int32)
    src = eidx[:, 0]
    dst = eidx[:, 1]
    psum, cnt = _make_sc_call()(table2, src, dst)
    out = _combine(psum.reshape(NC, N_PAD, HFEAT),
                   cnt.reshape(NC, N_PAD, CNTW))
    return out[:NUM_MESH][None]


# speedup vs baseline: 1.2030x; 1.2030x over previous
import functools

import jax
import jax.numpy as jnp
from jax import lax
from jax.experimental import pallas as pl
from jax.experimental.pallas import tpu as pltpu
from jax.experimental.pallas import tpu_sc as plsc

NUM_MESH = 10242
FEAT = 128
HFEAT = FEAT // 2
CNTW = 16
N_PAD = 10368
NC = 2
NS = 16
EDGES = 320000
CHUNK = 80
NBUF = 2
E_PAD = 320000
E_PER_T = E_PAD // NS
N_CHUNKS = E_PER_T // CHUNK
N_OUTER = N_CHUNKS // NBUF
ROWS_PER_TILE = N_PAD // NS
STG = 24


def _sc_body(table_hbm, src_hbm, dst_hbm,
             psum_hbm, cnt_hbm,
             src_v, dst_v, idx_v, rows_v, ones_v, stg_v, cstg_v,
             acc_sh, cnt_sh, sem_in, sem_g):
    c = lax.axis_index("c")
    s = lax.axis_index("s")
    row0 = s * ROWS_PER_TILE
    core0 = c == 0
    core1 = c == 1

    zeros16 = jnp.zeros((16,), jnp.float32)
    ones16 = jnp.ones((16,), jnp.float32)

    def fill_stg(i, carry):
        for j in range(HFEAT // 16):
            stg_v[i, pl.ds(j * 16, 16)] = zeros16
        return carry
    lax.fori_loop(0, STG, fill_stg, 0)

    def fill_cstg(i, carry):
        cstg_v[i, pl.ds(0, 16)] = zeros16
        return carry
    lax.fori_loop(0, STG, fill_cstg, 0)

    def fill_ones(i, carry):
        ones_v[i, pl.ds(0, 16)] = ones16
        return carry
    lax.fori_loop(0, CHUNK, fill_ones, 0)

    for q in range(ROWS_PER_TILE // STG):
        pltpu.sync_copy(stg_v, acc_sh.at[pl.ds(row0 + q * STG, STG)])
        pltpu.sync_copy(cstg_v, cnt_sh.at[pl.ds(row0 + q * STG, STG)])

    plsc.subcore_barrier()

    base_t = s * E_PER_T

    def issue_loads(k, b):
        base = pl.multiple_of(base_t + k * CHUNK, 8)
        pltpu.async_copy(src_hbm.at[pl.ds(base, CHUNK)], src_v.at[b], sem_in[b])
        pltpu.async_copy(dst_hbm.at[pl.ds(base, CHUNK)], dst_v.at[b], sem_in[b])

    def wait_loads(b):
        pltpu.make_async_copy(src_hbm.at[pl.ds(0, CHUNK)], src_v.at[b],
                              sem_in[b]).wait()
        pltpu.make_async_copy(dst_hbm.at[pl.ds(0, CHUNK)], dst_v.at[b],
                              sem_in[b]).wait()

    def transform_and_gather(b):
        for j in range(CHUNK // 16):
            sl = pl.ds(j * 16, 16)
            idx_v[b, sl] = src_v[b, sl] * 2 + c
        pltpu.async_copy(table_hbm.at[idx_v.at[b]], rows_v.at[b], sem_g[b])

    def wait_gather(b):
        pltpu.make_async_copy(table_hbm.at[idx_v.at[b]], rows_v.at[b],
                              sem_g[b]).wait()

    issue_loads(0, 0)
    issue_loads(1, 1)
    wait_loads(0)
    transform_and_gather(0)

    def outer_body(k0, carry):
        for b in range(NBUF):
            k = k0 * NBUF + b
            nb = (b + 1) % NBUF
            if b < NBUF - 1:
                wait_loads(nb)
                transform_and_gather(nb)
            else:
                @pl.when(k0 < N_OUTER - 1)
                def _():
                    wait_loads(nb)
                    transform_and_gather(nb)

            wait_gather(b)
            pltpu.sync_copy(rows_v.at[b], acc_sh.at[dst_v.at[b]], add=True)

            @pl.when(core0 if b == 0 else core1)
            def _():
                pltpu.sync_copy(ones_v, cnt_sh.at[dst_v.at[b]], add=True)

            @pl.when(k0 < N_OUTER - 1)
            def _():
                issue_loads(k + NBUF, b)
        return carry

    lax.fori_loop(0, N_OUTER, outer_body, 0)

    plsc.subcore_barrier()

    out0 = c * N_PAD + row0
    for q in range(ROWS_PER_TILE // STG):
        pltpu.sync_copy(acc_sh.at[pl.ds(row0 + q * STG, STG)], stg_v)
        pltpu.sync_copy(stg_v, psum_hbm.at[pl.ds(out0 + q * STG, STG)])
        pltpu.sync_copy(cnt_sh.at[pl.ds(row0 + q * STG, STG)], cstg_v)
        pltpu.sync_copy(cstg_v, cnt_hbm.at[pl.ds(out0 + q * STG, STG)])


def _make_sc_call():
    mesh = plsc.VectorSubcoreMesh(core_axis_name="c", subcore_axis_name="s")
    return functools.partial(
        pl.kernel,
        mesh=mesh,
        compiler_params=pltpu.CompilerParams(use_tc_tiling_on_sc=False),
        out_type=(
            jax.ShapeDtypeStruct((NC * N_PAD, HFEAT), jnp.float32),
            jax.ShapeDtypeStruct((NC * N_PAD, CNTW), jnp.float32),
        ),
        scratch_types=[
            pltpu.VMEM((NBUF, CHUNK), jnp.int32),
            pltpu.VMEM((NBUF, CHUNK), jnp.int32),
            pltpu.VMEM((NBUF, CHUNK), jnp.int32),
            pltpu.VMEM((NBUF, CHUNK, HFEAT), jnp.float32),
            pltpu.VMEM((CHUNK, CNTW), jnp.float32),
            pltpu.VMEM((STG, HFEAT), jnp.float32),
            pltpu.VMEM((STG, CNTW), jnp.float32),
            pltpu.VMEM_SHARED((N_PAD, HFEAT), jnp.float32),
            pltpu.VMEM_SHARED((N_PAD, CNTW), jnp.float32),
            [pltpu.SemaphoreType.DMA] * NBUF,
            [pltpu.SemaphoreType.DMA] * NBUF,
        ],
    )(_sc_body)


def _combine_body(psum_ref, cnt_ref, out_ref):
    total = jnp.concatenate((psum_ref[0], psum_ref[1]), axis=1)
    counts = cnt_ref[0, :, 0] + cnt_ref[1, :, 0]
    out_ref[...] = total / jnp.maximum(counts, 1.0)[:, None]


def _combine(psum, cnt):
    grid = N_PAD // 128
    return pl.pallas_call(
        _combine_body,
        grid=(grid,),
        in_specs=[
            pl.BlockSpec((NC, 128, HFEAT), lambda i: (0, i, 0)),
            pl.BlockSpec((NC, 128, CNTW), lambda i: (0, i, 0)),
        ],
        out_specs=pl.BlockSpec((128, FEAT), lambda i: (i, 0)),
        out_shape=jax.ShapeDtypeStruct((N_PAD, FEAT), jnp.float32),
    )(psum, cnt)


def kernel(grid_node_features, edge_index):
    feats = grid_node_features[0]
    table2 = feats.reshape(-1, HFEAT)
    eidx = edge_index[0].astype(jnp.int32)
    src = eidx[:, 0]
    dst = eidx[:, 1]
    psum, cnt = _make_sc_call()(table2, src, dst)
    out = _combine(psum.reshape(NC, N_PAD, HFEAT),
                   cnt.reshape(NC, N_PAD, CNTW))
    return out[:NUM_MESH][None]


# speedup vs baseline: 1.2314x; 1.0236x over previous
import functools

import jax
import jax.numpy as jnp
from jax import lax
from jax.experimental import pallas as pl
from jax.experimental.pallas import tpu as pltpu
from jax.experimental.pallas import tpu_sc as plsc

NUM_MESH = 10242
FEAT = 128
HFEAT = FEAT // 2
CNTW = 16
N_PAD = 10368
NC = 2
NS = 16
EDGES = 320000
CHUNK = 80
NBUF = 2
E_PAD = 320000
E_PER_T = E_PAD // NS
N_CHUNKS = E_PER_T // CHUNK
N_OUTER = N_CHUNKS // NBUF
ROWS_PER_TILE = N_PAD // NS
STG = 72


def _sc_body(table_hbm, src_hbm, dst_hbm,
             psum_hbm, cnt_hbm,
             src_v, dst_v, idx_v, rows_v, ones_v, stg_v, cstg_v,
             acc_sh, cnt_sh, sem_in, sem_g):
    c = lax.axis_index("c")
    s = lax.axis_index("s")
    row0 = s * ROWS_PER_TILE
    core0 = c == 0
    core1 = c == 1

    zeros16 = jnp.zeros((16,), jnp.float32)
    ones16 = jnp.ones((16,), jnp.float32)

    def fill_stg(i, carry):
        for j in range(HFEAT // 16):
            stg_v[i, pl.ds(j * 16, 16)] = zeros16
        return carry
    lax.fori_loop(0, STG, fill_stg, 0)

    def fill_cstg(i, carry):
        cstg_v[i, pl.ds(0, 16)] = zeros16
        return carry
    lax.fori_loop(0, STG, fill_cstg, 0)

    def fill_ones(i, carry):
        ones_v[i, pl.ds(0, 16)] = ones16
        return carry
    lax.fori_loop(0, CHUNK, fill_ones, 0)

    for q in range(ROWS_PER_TILE // STG):
        pltpu.sync_copy(stg_v, acc_sh.at[pl.ds(row0 + q * STG, STG)])
        pltpu.sync_copy(cstg_v, cnt_sh.at[pl.ds(row0 + q * STG, STG)])

    plsc.subcore_barrier()

    base_t = s * E_PER_T

    def issue_loads(k, b):
        base = pl.multiple_of(base_t + k * CHUNK, 8)
        pltpu.async_copy(src_hbm.at[pl.ds(base, CHUNK)], src_v.at[b], sem_in[b])
        pltpu.async_copy(dst_hbm.at[pl.ds(base, CHUNK)], dst_v.at[b], sem_in[b])

    def wait_loads(b):
        pltpu.make_async_copy(src_hbm.at[pl.ds(0, CHUNK)], src_v.at[b],
                              sem_in[b]).wait()
        pltpu.make_async_copy(dst_hbm.at[pl.ds(0, CHUNK)], dst_v.at[b],
                              sem_in[b]).wait()

    def transform_and_gather(b):
        for j in range(CHUNK // 16):
            sl = pl.ds(j * 16, 16)
            idx_v[b, sl] = src_v[b, sl] * 2 + c
        pltpu.async_copy(table_hbm.at[idx_v.at[b]], rows_v.at[b], sem_g[b])

    def wait_gather(b):
        pltpu.make_async_copy(table_hbm.at[idx_v.at[b]], rows_v.at[b],
                              sem_g[b]).wait()

    issue_loads(0, 0)
    issue_loads(1, 1)
    wait_loads(0)
    transform_and_gather(0)

    def outer_body(k0, carry):
        for b in range(NBUF):
            k = k0 * NBUF + b
            nb = (b + 1) % NBUF
            if b < NBUF - 1:
                wait_loads(nb)
                transform_and_gather(nb)
            else:
                @pl.when(k0 < N_OUTER - 1)
                def _():
                    wait_loads(nb)
                    transform_and_gather(nb)

            wait_gather(b)
            pltpu.sync_copy(rows_v.at[b], acc_sh.at[dst_v.at[b]], add=True)

            @pl.when(core0 if b == 0 else core1)
            def _():
                pltpu.sync_copy(ones_v, cnt_sh.at[dst_v.at[b]], add=True)

            @pl.when(k0 < N_OUTER - 1)
            def _():
                issue_loads(k + NBUF, b)
        return carry

    lax.fori_loop(0, N_OUTER, outer_body, 0)

    plsc.subcore_barrier()

    out0 = c * N_PAD + row0
    for q in range(ROWS_PER_TILE // STG):
        pltpu.sync_copy(acc_sh.at[pl.ds(row0 + q * STG, STG)], stg_v)
        pltpu.sync_copy(stg_v, psum_hbm.at[pl.ds(out0 + q * STG, STG)])
        pltpu.sync_copy(cnt_sh.at[pl.ds(row0 + q * STG, STG)], cstg_v)
        pltpu.sync_copy(cstg_v, cnt_hbm.at[pl.ds(out0 + q * STG, STG)])


def _make_sc_call():
    mesh = plsc.VectorSubcoreMesh(core_axis_name="c", subcore_axis_name="s")
    return functools.partial(
        pl.kernel,
        mesh=mesh,
        compiler_params=pltpu.CompilerParams(use_tc_tiling_on_sc=False),
        out_type=(
            jax.ShapeDtypeStruct((NC * N_PAD, HFEAT), jnp.float32),
            jax.ShapeDtypeStruct((NC * N_PAD, CNTW), jnp.float32),
        ),
        scratch_types=[
            pltpu.VMEM((NBUF, CHUNK), jnp.int32),
            pltpu.VMEM((NBUF, CHUNK), jnp.int32),
            pltpu.VMEM((NBUF, CHUNK), jnp.int32),
            pltpu.VMEM((NBUF, CHUNK, HFEAT), jnp.float32),
            pltpu.VMEM((CHUNK, CNTW), jnp.float32),
            pltpu.VMEM((STG, HFEAT), jnp.float32),
            pltpu.VMEM((STG, CNTW), jnp.float32),
            pltpu.VMEM_SHARED((N_PAD, HFEAT), jnp.float32),
            pltpu.VMEM_SHARED((N_PAD, CNTW), jnp.float32),
            [pltpu.SemaphoreType.DMA] * NBUF,
            [pltpu.SemaphoreType.DMA] * NBUF,
        ],
    )(_sc_body)


def _combine_body(psum_ref, cnt_ref, out_ref):
    total = jnp.concatenate((psum_ref[0], psum_ref[1]), axis=1)
    counts = cnt_ref[0, :, 0] + cnt_ref[1, :, 0]
    out_ref[...] = total / jnp.maximum(counts, 1.0)[:, None]


def _combine(psum, cnt):
    grid = N_PAD // 128
    return pl.pallas_call(
        _combine_body,
        grid=(grid,),
        in_specs=[
            pl.BlockSpec((NC, 128, HFEAT), lambda i: (0, i, 0)),
            pl.BlockSpec((NC, 128, CNTW), lambda i: (0, i, 0)),
        ],
        out_specs=pl.BlockSpec((128, FEAT), lambda i: (i, 0)),
        out_shape=jax.ShapeDtypeStruct((N_PAD, FEAT), jnp.float32),
    )(psum, cnt)


def kernel(grid_node_features, edge_index):
    feats = grid_node_features[0]
    table2 = feats.reshape(-1, HFEAT)
    eidx = edge_index[0].astype(jnp.int32)
    src = eidx[:, 0]
    dst = eidx[:, 1]
    psum, cnt = _make_sc_call()(table2, src, dst)
    out = _combine(psum.reshape(NC, N_PAD, HFEAT),
                   cnt.reshape(NC, N_PAD, CNTW))
    return out[:NUM_MESH][None]
